# Initial kernel scaffold; baseline (speedup 1.0000x reference)
import jax
import jax.numpy as jnp
from jax.experimental import pallas as pl


def kernel(x, edge_index, batch, W1, b1, Ws1, bs1, W2, b2, Ws2, bs2, W3, b3, Ws3, bs3):
    raise NotImplementedError("write your pallas kernel here")


# speedup vs baseline: 1.8920x; 1.8920x over previous
import functools
import math

import jax
import jax.numpy as jnp
from jax.experimental import pallas as pl

N = 10000
F = 128
RATIO = 0.5


def _mm_body(h_ref, w_ref, b_ref, o_ref):
    o_ref[...] = jnp.dot(h_ref[...], w_ref[...],
                         preferred_element_type=jnp.float32) + b_ref[...]


@functools.partial(jax.jit, static_argnames=())
def _matmul(h, W, b):
    blk = 2000
    return pl.pallas_call(
        _mm_body,
        grid=(N // blk,),
        in_specs=[
            pl.BlockSpec((blk, F), lambda i: (i, 0)),
            pl.BlockSpec((F, W.shape[1]), lambda i: (0, 0)),
            pl.BlockSpec((1, W.shape[1]), lambda i: (0, 0)),
        ],
        out_specs=pl.BlockSpec((blk, W.shape[1]), lambda i: (i, 0)),
        out_shape=jax.ShapeDtypeStruct((N, W.shape[1]), jnp.float32),
    )(h, W, b[None, :])


def kernel(x, edge_index, batch, W1, b1, Ws1, bs1, W2, b2, Ws2, bs2, W3, b3, Ws3, bs3):
    src = edge_index[0]
    dst = edge_index[1]
    k1 = math.ceil(RATIO * N)
    k2 = math.ceil(RATIO * k1)
    k3 = math.ceil(RATIO * k2)
    m = jnp.ones((N,), jnp.float32)
    h = x
    outs = []
    for (W, b, Ws, bs, k) in ((W1, b1, Ws1, bs1, k1), (W2, b2, Ws2, bs2, k2),
                              (W3, b3, Ws3, bs3, k3)):
        valid = (m[src] > 0) & (m[dst] > 0)
        srcp = jnp.where(valid, src, N)
        vf = valid.astype(jnp.float32)
        deg = jax.ops.segment_sum(vf, dst, num_segments=N) + m
        dis = jnp.where(deg > 0, 1.0 / jnp.sqrt(jnp.maximum(deg, 1e-12)), 0.0)
        hW = _matmul(h, W, b * 0.0)
        g = jnp.concatenate([dis[:, None] * hW, jnp.zeros((1, F), hW.dtype)], 0)
        S = jax.ops.segment_sum(g[srcp], dst, num_segments=N)
        h2 = jax.nn.relu(dis[:, None] * S + (dis * dis * m)[:, None] * hW + b)
        hs = (h2 @ Ws)[:, 0]
        gs = jnp.concatenate([dis * hs, jnp.zeros((1,), hs.dtype)], 0)
        Ss = jax.ops.segment_sum(gs[srcp], dst, num_segments=N)
        score = dis * Ss + dis * dis * m * hs + bs[0]
        sm = jnp.where(m > 0, score, -jnp.inf)
        vals, perm = jax.lax.top_k(sm, k)
        m_new = jnp.zeros((N,), jnp.float32).at[perm].set(1.0)
        h = h2 * jnp.tanh(score)[:, None] * m_new[:, None]
        gmax = jnp.max(jnp.where(m_new[:, None] > 0, h, -jnp.inf), axis=0)
        gmean = jnp.sum(h * m_new[:, None], axis=0) / k
        outs.append(jnp.concatenate([gmax, gmean])[None, :])
        m = m_new
    return outs[0] + outs[1] + outs[2]


# speedup vs baseline: 34.6390x; 18.3077x over previous
import functools
import math

import jax
import jax.numpy as jnp
from jax import lax
from jax.experimental import pallas as pl
from jax.experimental.pallas import tpu as pltpu
from jax.experimental.pallas import tpu_sc as plsc

N = 10000
NP = 10240
F = 128
E = 320000
RATIO = 0.5
NC, NS = 2, 16
NW = NC * NS
EPW = E // NW
EB = 125
NB = EPW // EB
RPT = NP // NS


def _make_segsum(width):
    mesh = plsc.VectorSubcoreMesh(core_axis_name="c", subcore_axis_name="s")

    def body(table, src3, dst3, zrows, out, src_v, dst_v, rows_v, acc, sem):
        c = lax.axis_index("c")
        s = lax.axis_index("s")
        w = c * NS + s
        pltpu.sync_copy(src3.at[w], src_v)
        pltpu.sync_copy(dst3.at[w], dst_v)
        pltpu.sync_copy(zrows, acc.at[pl.ds(s * RPT, RPT)])
        plsc.subcore_barrier()

        def step(j, _):
            pltpu.async_copy(table.at[src_v.at[j]], rows_v, sem).wait()
            pltpu.sync_copy(rows_v, acc.at[dst_v.at[j]], add=True)
            return _

        lax.fori_loop(0, NB, step, 0, unroll=False)
        plsc.subcore_barrier()
        pltpu.sync_copy(acc.at[pl.ds(s * RPT, RPT)],
                        out.at[c, pl.ds(s * RPT, RPT)])

    return pl.kernel(
        body,
        out_type=jax.ShapeDtypeStruct((NC, NP, width), jnp.float32),
        mesh=mesh,
        scratch_types=[
            pltpu.VMEM((NB, EB), jnp.int32),
            pltpu.VMEM((NB, EB), jnp.int32),
            pltpu.VMEM((EB, width), jnp.float32),
            pltpu.VMEM_SHARED((NP, width), jnp.float32),
            pltpu.SemaphoreType.DMA,
        ],
        compiler_params=pltpu.CompilerParams(use_tc_tiling_on_sc=False),
    )


_segsum128 = _make_segsum(F)
_segsum16 = _make_segsum(16)


def _sc_segsum(table, src3, dst3, zrows):
    width = table.shape[1]
    fn = _segsum128 if width == F else _segsum16
    part = fn(table, src3, dst3, zrows)
    return part[0] + part[1]


def _mm_body(h_ref, w_ref, o_ref):
    o_ref[...] = jnp.dot(h_ref[...], w_ref[...],
                         preferred_element_type=jnp.float32)


def _matmul(h, W):
    blk = 2000
    return pl.pallas_call(
        _mm_body,
        grid=(N // blk,),
        in_specs=[
            pl.BlockSpec((blk, F), lambda i: (i, 0)),
            pl.BlockSpec((F, W.shape[1]), lambda i: (0, 0)),
        ],
        out_specs=pl.BlockSpec((blk, W.shape[1]), lambda i: (i, 0)),
        out_shape=jax.ShapeDtypeStruct((N, W.shape[1]), jnp.float32),
    )(h, W)


def kernel(x, edge_index, batch, W1, b1, Ws1, bs1, W2, b2, Ws2, bs2, W3, b3, Ws3, bs3):
    src = edge_index[0]
    dst = edge_index[1]
    src3 = src.reshape(NW, NB, EB)
    dst3 = dst.reshape(NW, NB, EB)
    z16 = jnp.zeros((RPT, 16), jnp.float32)
    z128 = jnp.zeros((RPT, F), jnp.float32)

    k1 = math.ceil(RATIO * N)
    k2 = math.ceil(RATIO * k1)
    k3 = math.ceil(RATIO * k2)
    m = jnp.ones((N,), jnp.float32)
    h = x
    outs = []
    for (W, b, Ws, bs, k) in ((W1, b1, Ws1, bs1, k1), (W2, b2, Ws2, bs2, k2),
                              (W3, b3, Ws3, bs3, k3)):
        m16 = jnp.pad(jnp.broadcast_to(m[:, None], (N, 16)), ((0, NP - N), (0, 0)))
        deg = _sc_segsum(m16, src3, dst3, z16)[:N, 0] + m
        dis = jnp.where(deg > 0, lax.rsqrt(jnp.maximum(deg, 1e-12)), 0.0)
        hW = _matmul(h, W)
        g = jnp.pad((dis * m)[:, None] * hW, ((0, NP - N), (0, 0)))
        S = _sc_segsum(g, src3, dst3, z128)[:N]
        h2 = jax.nn.relu(dis[:, None] * S + (dis * dis * m)[:, None] * hW + b)
        hs = (h2 @ Ws)[:, 0]
        gs16 = jnp.pad(jnp.broadcast_to((dis * m * hs)[:, None], (N, 16)),
                       ((0, NP - N), (0, 0)))
        Ss = _sc_segsum(gs16, src3, dst3, z16)[:N, 0]
        score = dis * Ss + dis * dis * m * hs + bs[0]
        sm = jnp.where(m > 0, score, -jnp.inf)
        vals, perm = jax.lax.top_k(sm, k)
        m_new = jnp.zeros((N,), jnp.float32).at[perm].set(1.0)
        h = h2 * jnp.tanh(score)[:, None] * m_new[:, None]
        gmax = jnp.max(jnp.where(m_new[:, None] > 0, h, -jnp.inf), axis=0)
        gmean = jnp.sum(h * m_new[:, None], axis=0) / k
        outs.append(jnp.concatenate([gmax, gmean])[None, :])
        m = m_new
    return outs[0] + outs[1] + outs[2]


# speedup vs baseline: 47.6571x; 1.3758x over previous
import functools
import math

import jax
import jax.numpy as jnp
from jax import lax
from jax.experimental import pallas as pl
from jax.experimental.pallas import tpu as pltpu
from jax.experimental.pallas import tpu_sc as plsc

N = 10000
NP = 10240
F = 128
E = 320000
RATIO = 0.5
NC, NS = 2, 16
NW = NC * NS
EPW = E // NW
RPT = NP // NS


def _make_segsum(width, eb):
    mesh = plsc.VectorSubcoreMesh(core_axis_name="c", subcore_axis_name="s")
    NB = EPW // eb

    def body(table, src3, dst3, zrows, out,
             src_v, dst_v, rows0, rows1, acc, sem0, sem1):
        c = lax.axis_index("c")
        s = lax.axis_index("s")
        w = c * NS + s
        pltpu.sync_copy(src3.at[w], src_v)
        pltpu.sync_copy(dst3.at[w], dst_v)
        pltpu.sync_copy(zrows, acc.at[pl.ds(s * RPT, RPT)])
        plsc.subcore_barrier()

        pltpu.async_copy(table.at[src_v.at[0]], rows0, sem0)

        def step(jj, _):
            j0 = jj * 2
            pltpu.async_copy(table.at[src_v.at[j0 + 1]], rows1, sem1)
            pltpu.make_async_copy(table.at[src_v.at[0]], rows0, sem0).wait()
            pltpu.sync_copy(rows0, acc.at[dst_v.at[j0]], add=True)

            @pl.when(j0 + 2 < NB)
            def _fire():
                pltpu.async_copy(table.at[src_v.at[j0 + 2]], rows0, sem0)

            pltpu.make_async_copy(table.at[src_v.at[0]], rows1, sem1).wait()
            pltpu.sync_copy(rows1, acc.at[dst_v.at[j0 + 1]], add=True)
            return _

        lax.fori_loop(0, NB // 2, step, 0, unroll=False)
        if NB % 2 == 1:
            pltpu.make_async_copy(table.at[src_v.at[0]], rows0, sem0).wait()
            pltpu.sync_copy(rows0, acc.at[dst_v.at[NB - 1]], add=True)
        plsc.subcore_barrier()
        pltpu.sync_copy(acc.at[pl.ds(s * RPT, RPT)],
                        out.at[c, pl.ds(s * RPT, RPT)])

    return pl.kernel(
        body,
        out_type=jax.ShapeDtypeStruct((NC, NP, width), jnp.float32),
        mesh=mesh,
        scratch_types=[
            pltpu.VMEM((NB, eb), jnp.int32),
            pltpu.VMEM((NB, eb), jnp.int32),
            pltpu.VMEM((eb, width), jnp.float32),
            pltpu.VMEM((eb, width), jnp.float32),
            pltpu.VMEM_SHARED((NP, width), jnp.float32),
            pltpu.SemaphoreType.DMA,
            pltpu.SemaphoreType.DMA,
        ],
        compiler_params=pltpu.CompilerParams(use_tc_tiling_on_sc=False),
    )


EB128 = 80
EB16 = 125
_segsum128 = _make_segsum(F, EB128)
_segsum16 = _make_segsum(16, EB16)


def _sc_segsum(table, src3, dst3, zrows):
    width = table.shape[1]
    fn = _segsum128 if width == F else _segsum16
    part = fn(table, src3, dst3, zrows)
    return part[0] + part[1]


def _mm_body(h_ref, w_ref, o_ref):
    o_ref[...] = jnp.dot(h_ref[...], w_ref[...],
                         preferred_element_type=jnp.float32)


def _matmul(h, W):
    blk = 2000
    return pl.pallas_call(
        _mm_body,
        grid=(N // blk,),
        in_specs=[
            pl.BlockSpec((blk, F), lambda i: (i, 0)),
            pl.BlockSpec((F, W.shape[1]), lambda i: (0, 0)),
        ],
        out_specs=pl.BlockSpec((blk, W.shape[1]), lambda i: (i, 0)),
        out_shape=jax.ShapeDtypeStruct((N, W.shape[1]), jnp.float32),
    )(h, W)


def kernel(x, edge_index, batch, W1, b1, Ws1, bs1, W2, b2, Ws2, bs2, W3, b3, Ws3, bs3):
    src = edge_index[0]
    dst = edge_index[1]
    src3a = src.reshape(NW, EPW // EB128, EB128)
    dst3a = dst.reshape(NW, EPW // EB128, EB128)
    src3b = src.reshape(NW, EPW // EB16, EB16)
    dst3b = dst.reshape(NW, EPW // EB16, EB16)
    z16 = jnp.zeros((RPT, 16), jnp.float32)
    z128 = jnp.zeros((RPT, F), jnp.float32)

    k1 = math.ceil(RATIO * N)
    k2 = math.ceil(RATIO * k1)
    k3 = math.ceil(RATIO * k2)
    m = jnp.ones((N,), jnp.float32)
    h = x
    outs = []
    for (W, b, Ws, bs, k) in ((W1, b1, Ws1, bs1, k1), (W2, b2, Ws2, bs2, k2),
                              (W3, b3, Ws3, bs3, k3)):
        m16 = jnp.pad(jnp.broadcast_to(m[:, None], (N, 16)), ((0, NP - N), (0, 0)))
        deg = _sc_segsum(m16, src3b, dst3b, z16)[:N, 0] + m
        dis = jnp.where(deg > 0, lax.rsqrt(jnp.maximum(deg, 1e-12)), 0.0)
        hW = _matmul(h, W)
        g = jnp.pad((dis * m)[:, None] * hW, ((0, NP - N), (0, 0)))
        S = _sc_segsum(g, src3a, dst3a, z128)[:N]
        h2 = jax.nn.relu(dis[:, None] * S + (dis * dis * m)[:, None] * hW + b)
        hs = (h2 @ Ws)[:, 0]
        gs16 = jnp.pad(jnp.broadcast_to((dis * m * hs)[:, None], (N, 16)),
                       ((0, NP - N), (0, 0)))
        Ss = _sc_segsum(gs16, src3b, dst3b, z16)[:N, 0]
        score = dis * Ss + dis * dis * m * hs + bs[0]
        sm = jnp.where(m > 0, score, -jnp.inf)
        vals, perm = jax.lax.top_k(sm, k)
        m_new = jnp.zeros((N,), jnp.float32).at[perm].set(1.0)
        h = h2 * jnp.tanh(score)[:, None] * m_new[:, None]
        gmax = jnp.max(jnp.where(m_new[:, None] > 0, h, -jnp.inf), axis=0)
        gmean = jnp.sum(h * m_new[:, None], axis=0) / k
        outs.append(jnp.concatenate([gmax, gmean])[None, :])
        m = m_new
    return outs[0] + outs[1] + outs[2]


# speedup vs baseline: 57.5101x; 1.2067x over previous
import functools
import math

import jax
import jax.numpy as jnp
from jax import lax
from jax.experimental import pallas as pl
from jax.experimental.pallas import tpu as pltpu
from jax.experimental.pallas import tpu_sc as plsc

N = 10000
NP = 10240
F = 128
E = 320000
RATIO = 0.5
NC, NS = 2, 16
NW = NC * NS
EPW = E // NW
RPT = NP // NS


def _make_segsum(width, eb):
    mesh = plsc.VectorSubcoreMesh(core_axis_name="c", subcore_axis_name="s")
    NB = EPW // eb

    def body(table, src3, dst3, zrows, out,
             src_v, dst_v, rows0, rows1, acc, sem0, sem1):
        c = lax.axis_index("c")
        s = lax.axis_index("s")
        w = c * NS + s
        pltpu.sync_copy(src3.at[w], src_v)
        pltpu.sync_copy(dst3.at[w], dst_v)
        pltpu.sync_copy(zrows, acc.at[pl.ds(s * RPT, RPT)])
        plsc.subcore_barrier()

        pltpu.async_copy(table.at[src_v.at[0]], rows0, sem0)

        def step(jj, _):
            j0 = jj * 2
            pltpu.async_copy(table.at[src_v.at[j0 + 1]], rows1, sem1)
            pltpu.make_async_copy(table.at[src_v.at[0]], rows0, sem0).wait()
            pltpu.sync_copy(rows0, acc.at[dst_v.at[j0]], add=True)

            @pl.when(j0 + 2 < NB)
            def _fire():
                pltpu.async_copy(table.at[src_v.at[j0 + 2]], rows0, sem0)

            pltpu.make_async_copy(table.at[src_v.at[0]], rows1, sem1).wait()
            pltpu.sync_copy(rows1, acc.at[dst_v.at[j0 + 1]], add=True)
            return _

        lax.fori_loop(0, NB // 2, step, 0, unroll=False)
        if NB % 2 == 1:
            pltpu.make_async_copy(table.at[src_v.at[0]], rows0, sem0).wait()
            pltpu.sync_copy(rows0, acc.at[dst_v.at[NB - 1]], add=True)
        plsc.subcore_barrier()
        pltpu.sync_copy(acc.at[pl.ds(s * RPT, RPT)],
                        out.at[c, pl.ds(s * RPT, RPT)])

    return pl.kernel(
        body,
        out_type=jax.ShapeDtypeStruct((NC, NP, width), jnp.float32),
        mesh=mesh,
        scratch_types=[
            pltpu.VMEM((NB, eb), jnp.int32),
            pltpu.VMEM((NB, eb), jnp.int32),
            pltpu.VMEM((eb, width), jnp.float32),
            pltpu.VMEM((eb, width), jnp.float32),
            pltpu.VMEM_SHARED((NP, width), jnp.float32),
            pltpu.SemaphoreType.DMA,
            pltpu.SemaphoreType.DMA,
        ],
        compiler_params=pltpu.CompilerParams(use_tc_tiling_on_sc=False),
    )


EB128 = 80
EB16 = 125
_segsum128 = _make_segsum(F, EB128)
_segsum16 = _make_segsum(16, EB16)


def _sc_segsum(table, src3, dst3, zrows):
    width = table.shape[1]
    fn = _segsum128 if width == F else _segsum16
    part = fn(table, src3, dst3, zrows)
    return part[0] + part[1]


G = NP // 128
BLK = 1280


def _mm_body(h_ref, w_ref, o_ref):
    o_ref[...] = jnp.dot(h_ref[...], w_ref[...],
                         preferred_element_type=jnp.float32)


def _matmul(h, W):
    return pl.pallas_call(
        _mm_body,
        grid=(NP // BLK,),
        in_specs=[
            pl.BlockSpec((BLK, F), lambda i: (i, 0)),
            pl.BlockSpec((F, W.shape[1]), lambda i: (0, 0)),
        ],
        out_specs=pl.BlockSpec((BLK, W.shape[1]), lambda i: (i, 0)),
        out_shape=jax.ShapeDtypeStruct((NP, W.shape[1]), jnp.float32),
    )(h, W)


def _prep_body(deg_ref, m_ref, hW_ref, g_ref, dis_ref):
    deg = deg_ref[...]
    m = m_ref[...]
    dis = jnp.where(deg > 0, lax.rsqrt(jnp.maximum(deg, 1e-12)), 0.0)
    dis_ref[...] = dis
    g_ref[...] = jnp.broadcast_to(dis * m, (BLK, F)) * hW_ref[...]


def _prep_g(deg1, m1, hW):
    return pl.pallas_call(
        _prep_body,
        grid=(NP // BLK,),
        in_specs=[pl.BlockSpec((BLK, 1), lambda i: (i, 0)),
                  pl.BlockSpec((BLK, 1), lambda i: (i, 0)),
                  pl.BlockSpec((BLK, F), lambda i: (i, 0))],
        out_specs=[pl.BlockSpec((BLK, F), lambda i: (i, 0)),
                   pl.BlockSpec((BLK, 1), lambda i: (i, 0))],
        out_shape=[jax.ShapeDtypeStruct((NP, F), jnp.float32),
                   jax.ShapeDtypeStruct((NP, 1), jnp.float32)],
    )(deg1, m1, hW)


def _h2_body(S0_ref, S1_ref, dis_ref, m_ref, hW_ref, b_ref, Ws_ref,
             h2_ref, hs_ref, gs_ref):
    dis = dis_ref[...]
    m = m_ref[...]
    S = S0_ref[0] + S1_ref[0]
    h2 = jax.nn.relu(jnp.broadcast_to(dis, (BLK, F)) * S
                     + jnp.broadcast_to(dis * dis * m, (BLK, F)) * hW_ref[...]
                     + b_ref[...])
    h2_ref[...] = h2
    hs = jnp.dot(h2, Ws_ref[...], preferred_element_type=jnp.float32)
    hs_ref[...] = hs
    gs_ref[...] = dis * m * hs


def _h2_hs(Spart, dis1, m1, hW, b, Ws):
    return pl.pallas_call(
        _h2_body,
        grid=(NP // BLK,),
        in_specs=[pl.BlockSpec((1, BLK, F), lambda i: (0, i, 0)),
                  pl.BlockSpec((1, BLK, F), lambda i: (1, i, 0)),
                  pl.BlockSpec((BLK, 1), lambda i: (i, 0)),
                  pl.BlockSpec((BLK, 1), lambda i: (i, 0)),
                  pl.BlockSpec((BLK, F), lambda i: (i, 0)),
                  pl.BlockSpec((1, F), lambda i: (0, 0)),
                  pl.BlockSpec((F, 1), lambda i: (0, 0))],
        out_specs=[pl.BlockSpec((BLK, F), lambda i: (i, 0)),
                   pl.BlockSpec((BLK, 1), lambda i: (i, 0)),
                   pl.BlockSpec((BLK, 1), lambda i: (i, 0))],
        out_shape=[jax.ShapeDtypeStruct((NP, F), jnp.float32),
                   jax.ShapeDtypeStruct((NP, 1), jnp.float32),
                   jax.ShapeDtypeStruct((NP, 1), jnp.float32)],
    )(Spart, Spart, dis1, m1, hW, b[None, :], Ws)


def _topk_body(k, ss_ref, dis_ref, m_ref, hs_ref, bs_ref, mnew_ref, score_ref):
    dis = dis_ref[...]
    m = m_ref[...]
    hs = hs_ref[...]
    score = dis * ss_ref[...] + dis * dis * m * hs + bs_ref[0, 0]
    score_ref[...] = score
    bits = lax.bitcast_convert_type(score, jnp.uint32)
    key = jnp.where(bits >> 31 == 0, bits | jnp.uint32(0x80000000), ~bits)
    key = jnp.where(m > 0, key, jnp.uint32(0))

    def bit_step(i, p):
        t = p | (jnp.uint32(1) << (31 - i))
        c = jnp.sum(jnp.where(key >= t, 1, 0), dtype=jnp.int32)
        return jnp.where(c >= k, t, p)

    T = lax.fori_loop(0, 32, bit_step, jnp.uint32(0))
    cg = jnp.sum(jnp.where(key > T, 1, 0), dtype=jnp.int32)
    need = k - cg
    eq = key == T
    idx = lax.broadcasted_iota(jnp.int32, (G, 128), 0) * 128 + \
        lax.broadcasted_iota(jnp.int32, (G, 128), 1)

    def j_step(b, Jp):
        cand = Jp | (jnp.int32(1) << (13 - b))
        c2 = jnp.sum(jnp.where(eq & (idx < cand), 1, 0), dtype=jnp.int32)
        return jnp.where(c2 < need, cand, Jp)

    Jp = lax.fori_loop(0, 14, j_step, jnp.int32(0))
    sel = (key > T) | (eq & (idx <= Jp) & (need > 0))
    mnew_ref[...] = sel.astype(jnp.float32)


def _topk(ss2, dis2, m2, hs2, bs, k):
    return pl.pallas_call(
        functools.partial(_topk_body, k),
        in_specs=[pl.BlockSpec((G, 128), lambda: (0, 0))] * 4 +
                 [pl.BlockSpec((1, 1), lambda: (0, 0))],
        out_specs=[pl.BlockSpec((G, 128), lambda: (0, 0))] * 2,
        out_shape=[jax.ShapeDtypeStruct((G, 128), jnp.float32)] * 2,
    )(ss2, dis2, m2, hs2, bs)


def _pool_body(k, h2_ref, score_ref, mnew_ref, o3_ref, or_ref):
    sc = score_ref[...]
    mn = mnew_ref[...]
    h3 = h2_ref[...] * jnp.tanh(sc) * mn
    o3_ref[...] = h3
    gmax = jnp.max(jnp.where(mn > 0, h3, -3.0e38), axis=0, keepdims=True)
    gmean = jnp.sum(h3, axis=0, keepdims=True) / k
    or_ref[...] = jnp.concatenate([gmax, gmean], axis=0)


def _pool(h2, score1, mnew1, k):
    return pl.pallas_call(
        functools.partial(_pool_body, float(k)),
        in_specs=[pl.BlockSpec((NP, F), lambda: (0, 0)),
                  pl.BlockSpec((NP, 1), lambda: (0, 0)),
                  pl.BlockSpec((NP, 1), lambda: (0, 0))],
        out_specs=[pl.BlockSpec((NP, F), lambda: (0, 0)),
                   pl.BlockSpec((2, F), lambda: (0, 0))],
        out_shape=[jax.ShapeDtypeStruct((NP, F), jnp.float32),
                   jax.ShapeDtypeStruct((2, F), jnp.float32)],
    )(h2, score1, mnew1)


def kernel(x, edge_index, batch, W1, b1, Ws1, bs1, W2, b2, Ws2, bs2, W3, b3, Ws3, bs3):
    src = edge_index[0]
    dst = edge_index[1]
    src3a = src.reshape(NW, EPW // EB128, EB128)
    dst3a = dst.reshape(NW, EPW // EB128, EB128)
    src3b = src.reshape(NW, EPW // EB16, EB16)
    dst3b = dst.reshape(NW, EPW // EB16, EB16)
    z16 = jnp.zeros((RPT, 16), jnp.float32)
    z128 = jnp.zeros((RPT, F), jnp.float32)

    k1 = math.ceil(RATIO * N)
    k2 = math.ceil(RATIO * k1)
    k3 = math.ceil(RATIO * k2)
    m1 = jnp.concatenate([jnp.ones((N, 1), jnp.float32),
                          jnp.zeros((NP - N, 1), jnp.float32)])
    h = jnp.pad(x, ((0, NP - N), (0, 0)))
    outs = []
    for (W, b, Ws, bs, k) in ((W1, b1, Ws1, bs1, k1), (W2, b2, Ws2, bs2, k2),
                              (W3, b3, Ws3, bs3, k3)):
        hW = _matmul(h, W)
        m16 = jnp.broadcast_to(m1, (NP, 16))
        degp = _segsum16(m16, src3b, dst3b, z16)
        deg1 = (degp[0, :, :1] + degp[1, :, :1]) + m1
        g, dis1 = _prep_g(deg1, m1, hW)
        Spart = _segsum128(g, src3a, dst3a, z128)
        h2, hs1, gs1 = _h2_hs(Spart, dis1, m1, hW, b, Ws)
        gs16 = jnp.broadcast_to(gs1, (NP, 16))
        Ssp = _segsum16(gs16, src3b, dst3b, z16)
        ss2 = (Ssp[0, :, 0] + Ssp[1, :, 0]).reshape(G, 128)
        mnew2, score2 = _topk(ss2, dis1.reshape(G, 128), m1.reshape(G, 128),
                              hs1.reshape(G, 128), bs.reshape(1, 1), k)
        h, orow = _pool(h2, score2.reshape(NP, 1), mnew2.reshape(NP, 1), k)
        outs.append(orow.reshape(1, 2 * F))
        m1 = mnew2.reshape(NP, 1)
    return outs[0] + outs[1] + outs[2]
